# Initial kernel scaffold; baseline (speedup 1.0000x reference)
import jax
import jax.numpy as jnp
from jax import lax
from jax.experimental import pallas as pl
from jax.experimental.pallas import tpu as pltpu
from jax.experimental.pallas import tpu_sc as plsc

NC = 2
NS = 16
NW = NC * NS

GROUP = 128
G = 8


def _emb_body(idx_hbm, table_hbm, out_hbm, idx_v, rows_v, gsem):
    ng = idx_hbm.shape[0]
    gpw = ng // NW
    chunks = gpw // G
    wid = lax.axis_index("s") * NC + lax.axis_index("c")

    def chunk_body(i, _):
        g0 = wid * gpw + i * G
        pltpu.sync_copy(idx_hbm.at[pl.ds(g0, G)], idx_v)
        for j in range(G):
            pltpu.async_copy(table_hbm.at[idx_v.at[j]], rows_v.at[j], gsem)
        for j in range(G):
            pltpu.make_async_copy(table_hbm.at[idx_v.at[j]], rows_v.at[j],
                                  gsem).wait()
        pltpu.sync_copy(rows_v, out_hbm.at[pl.ds(g0, G)])
        return 0

    lax.fori_loop(0, chunks, chunk_body, 0)


def kernel(token_ids, embedding_table):
    b, s = token_ids.shape
    dim = embedding_table.shape[1]
    total = b * s
    ng = total // GROUP
    idx = token_ids.reshape(ng, GROUP)

    mesh = plsc.VectorSubcoreMesh(core_axis_name="c", subcore_axis_name="s",
                                  num_cores=NC, num_subcores=NS)
    out = pl.kernel(
        _emb_body,
        out_type=jax.ShapeDtypeStruct((ng, GROUP, dim), jnp.float32),
        mesh=mesh,
        scratch_types=[
            pltpu.VMEM((G, GROUP), jnp.int32),
            pltpu.VMEM((G, GROUP, dim), jnp.float32),
            pltpu.SemaphoreType.DMA,
        ],
    )(idx, embedding_table)
    return out.reshape(b, s, dim)


# speedup vs baseline: 4.8089x; 4.8089x over previous
import jax
import jax.numpy as jnp
from jax import lax
from jax.experimental import pallas as pl
from jax.experimental.pallas import tpu as pltpu
from jax.experimental.pallas import tpu_sc as plsc

NC = 2
NS = 16
NW = NC * NS

GROUP = 128
G = 8


def _emb_body(idx_hbm, table_hbm, out_hbm, idx_v, rows_v, gsem):
    ng = idx_hbm.shape[0]
    gpw = ng // NW
    chunks = gpw // G
    wid = lax.axis_index("s") * NC + lax.axis_index("c")

    def chunk_body(i, _):
        g0 = wid * gpw + i * G
        pltpu.sync_copy(idx_hbm.at[pl.ds(g0, G)], idx_v)
        for j in range(G):
            pltpu.async_copy(table_hbm.at[idx_v.at[j]], rows_v.at[j], gsem)
        for j in range(G):
            pltpu.make_async_copy(table_hbm.at[idx_v.at[j]], rows_v.at[j],
                                  gsem).wait()
        pltpu.sync_copy(rows_v, out_hbm.at[pl.ds(g0, G)])
        return 0

    lax.fori_loop(0, chunks, chunk_body, 0)


def kernel(token_ids, embedding_table):
    b, s = token_ids.shape
    dim = embedding_table.shape[1]
    total = b * s
    ng = total // GROUP
    idx = token_ids.reshape(ng, GROUP)

    mesh = plsc.VectorSubcoreMesh(core_axis_name="c", subcore_axis_name="s",
                                  num_cores=NC, num_subcores=NS)
    out = pl.kernel(
        _emb_body,
        out_type=jax.ShapeDtypeStruct((ng, GROUP, dim), jnp.float32),
        mesh=mesh,
        scratch_types=[
            pltpu.VMEM((G, GROUP), jnp.int32),
            pltpu.VMEM((G, GROUP, dim), jnp.float32),
            pltpu.SemaphoreType.DMA,
        ],
        compiler_params=pltpu.CompilerParams(use_tc_tiling_on_sc=False),
    )(idx, embedding_table)
    return out.reshape(b, s, dim)


# speedup vs baseline: 4.9042x; 1.0198x over previous
import jax
import jax.numpy as jnp
from jax import lax
from jax.experimental import pallas as pl
from jax.experimental.pallas import tpu as pltpu
from jax.experimental.pallas import tpu_sc as plsc

NC = 2
NS = 16
NW = NC * NS

R = 8


def _emb_body(idx_hbm, table_hbm, out_hbm, idx_v, rows_v, gsem):
    nb, sl = idx_hbm.shape
    rpw = nb // NW
    chunks = rpw // R
    wid = lax.axis_index("s") * NC + lax.axis_index("c")

    splits = []
    o = 0
    while o < sl:
        w = min(128, sl - o)
        splits.append((o, w))
        o += w

    def chunk_body(i, _):
        r0 = wid * rpw + i * R
        pltpu.sync_copy(idx_hbm.at[pl.ds(r0, R)], idx_v)
        for j in range(R):
            for (o, w) in splits:
                pltpu.async_copy(table_hbm.at[idx_v.at[j, pl.ds(o, w)]],
                                 rows_v.at[j, pl.ds(o, w)], gsem)
        for j in range(R):
            for (o, w) in splits:
                pltpu.make_async_copy(table_hbm.at[idx_v.at[j, pl.ds(o, w)]],
                                      rows_v.at[j, pl.ds(o, w)], gsem).wait()
        pltpu.sync_copy(rows_v, out_hbm.at[pl.ds(r0, R)])
        return 0

    lax.fori_loop(0, chunks, chunk_body, 0)


def kernel(token_ids, embedding_table):
    nb, sl = token_ids.shape
    dim = embedding_table.shape[1]

    mesh = plsc.VectorSubcoreMesh(core_axis_name="c", subcore_axis_name="s",
                                  num_cores=NC, num_subcores=NS)
    out = pl.kernel(
        _emb_body,
        out_type=jax.ShapeDtypeStruct((nb, sl, dim), jnp.float32),
        mesh=mesh,
        scratch_types=[
            pltpu.VMEM((R, sl), jnp.int32),
            pltpu.VMEM((R, sl, dim), jnp.float32),
            pltpu.SemaphoreType.DMA,
        ],
        compiler_params=pltpu.CompilerParams(use_tc_tiling_on_sc=False),
    )(token_ids, embedding_table)
    return out
